# baseline (speedup 1.0000x reference)
import jax
import jax.numpy as jnp
from jax.experimental import pallas as pl

_BB = 8


def _pwc_body(times_ref, vals_ref, t_ref, arr_ref, out_ref):
    tt = t_ref[0, 0]
    K = vals_ref.shape[-1]
    t_lo = times_ref[0, :K]
    t_hi = times_ref[0, 1:]
    mask = (t_lo <= tt) & (tt < t_hi)
    vals = jnp.where(mask[None, :], vals_ref[...], 0.0).sum(axis=1)
    out_ref[...] = vals[:, None, None] * arr_ref[...][None]


@jax.jit
def kernel(times, values, array, t):
    B, K = values.shape
    N = array.shape[0]

    return pl.pallas_call(
        _pwc_body,
        grid=(B // _BB,),
        in_specs=[
            pl.BlockSpec((1, K + 1), lambda b: (0, 0)),
            pl.BlockSpec((_BB, K), lambda b: (b, 0)),
            pl.BlockSpec((1, 1), lambda b: (0, 0)),
            pl.BlockSpec((N, N), lambda b: (0, 0)),
        ],
        out_specs=pl.BlockSpec((_BB, N, N), lambda b: (b, 0, 0)),
        out_shape=jax.ShapeDtypeStruct((B, N, N), jnp.float32),
    )(times.reshape(1, K + 1), values, t.reshape(1, 1), array)


# speedup vs baseline: 1.1516x; 1.1516x over previous
import jax
import jax.numpy as jnp
from jax.experimental import pallas as pl

_BB = 32


def _pwc_body(times_ref, vals_ref, t_ref, arr_ref, out_ref):
    tt = t_ref[0, 0]
    K = vals_ref.shape[-1]
    t_lo = times_ref[0, :K]
    t_hi = times_ref[0, 1:]
    mask = (t_lo <= tt) & (tt < t_hi)
    vals = jnp.where(mask[None, :], vals_ref[...], 0.0).sum(axis=1)
    out_ref[...] = vals[:, None, None] * arr_ref[...][None]


@jax.jit
def kernel(times, values, array, t):
    B, K = values.shape
    N = array.shape[0]

    return pl.pallas_call(
        _pwc_body,
        grid=(B // _BB,),
        in_specs=[
            pl.BlockSpec((1, K + 1), lambda b: (0, 0)),
            pl.BlockSpec((_BB, K), lambda b: (b, 0)),
            pl.BlockSpec((1, 1), lambda b: (0, 0)),
            pl.BlockSpec((N, N), lambda b: (0, 0)),
        ],
        out_specs=pl.BlockSpec((_BB, N, N), lambda b: (b, 0, 0)),
        out_shape=jax.ShapeDtypeStruct((B, N, N), jnp.float32),
    )(times.reshape(1, K + 1), values, t.reshape(1, 1), array)


# speedup vs baseline: 1.1797x; 1.0244x over previous
import jax
import jax.numpy as jnp
from jax.experimental import pallas as pl
from jax.experimental.pallas import tpu as pltpu

_BB = 16


def _pwc_body(times_ref, vals_ref, t_ref, arr_ref, out_ref):
    tt = t_ref[0, 0]
    K = vals_ref.shape[-1]
    t_lo = times_ref[0, :K]
    t_hi = times_ref[0, 1:]
    mask = (t_lo <= tt) & (tt < t_hi)
    vals = jnp.where(mask[None, :], vals_ref[...], 0.0).sum(axis=1)
    out_ref[...] = vals[:, None, None] * arr_ref[...][None]


@jax.jit
def kernel(times, values, array, t):
    B, K = values.shape
    N = array.shape[0]

    return pl.pallas_call(
        _pwc_body,
        grid=(B // _BB,),
        in_specs=[
            pl.BlockSpec((1, K + 1), lambda b: (0, 0)),
            pl.BlockSpec((_BB, K), lambda b: (b, 0)),
            pl.BlockSpec((1, 1), lambda b: (0, 0)),
            pl.BlockSpec((N, N), lambda b: (0, 0)),
        ],
        out_specs=pl.BlockSpec((_BB, N, N), lambda b: (b, 0, 0)),
        out_shape=jax.ShapeDtypeStruct((B, N, N), jnp.float32),
        compiler_params=pltpu.CompilerParams(
            dimension_semantics=(pltpu.GridDimensionSemantics.PARALLEL,)
        ),
    )(times.reshape(1, K + 1), values, t.reshape(1, 1), array)
